# baseline (speedup 1.0000x reference)
import functools

import jax
import jax.numpy as jnp
import numpy as np
from jax import lax
from jax.experimental import pallas as pl
from jax.experimental.pallas import tpu as pltpu
from jax.experimental.pallas import tpu_sc as plsc

_B = 262144
_N = 512
_F = 96
_HID = 64
_L_DIR = 4
_SCALE = 1.5

_NC = 2
_NS = 16
_NW = _NC * _NS
_LANES = 16
_PER_W = _B // _NW
_FP = 128
_CH = 128
_NCH = _PER_W // _CH

_BB = 1024


def _col(v):
    q = (v / (2.0 * _SCALE) + 0.5) * float(_N)
    qi = q.astype(jnp.int32)
    return jnp.clip(qi, 0, _N - 1)


def _tc_relayout(planeT):
    def body(in_r, out_r):
        t = in_r[0]
        tt = t.T
        out_r[:, 0:_F] = tt
        out_r[:, _F:_FP] = jnp.zeros((_N, _FP - _F), jnp.float32)

    return pl.pallas_call(
        body,
        grid=(_N,),
        in_specs=[pl.BlockSpec((1, _F, _N), lambda i: (i, 0, 0))],
        out_specs=pl.BlockSpec((_N, _FP), lambda i: (i, 0)),
        out_shape=jax.ShapeDtypeStruct((_N * _N, _FP), jnp.float32),
        compiler_params=pltpu.CompilerParams(
            dimension_semantics=("parallel",)),
    )(planeT)


def _sc_gather1(xa, xb, tab):
    mesh = plsc.VectorSubcoreMesh(core_axis_name="c", subcore_axis_name="s")

    @functools.partial(
        pl.kernel,
        mesh=mesh,
        out_type=jax.ShapeDtypeStruct((_B, _FP), jnp.float32),
        scratch_types=[
            pltpu.VMEM((2, _CH), jnp.float32),
            pltpu.VMEM((2, _CH), jnp.float32),
            pltpu.VMEM((2, _CH), jnp.int32),
            pltpu.VMEM((_CH, _FP), jnp.float32),
            pltpu.VMEM((_CH, _FP), jnp.float32),
            pltpu.SemaphoreType.DMA,
            pltpu.SemaphoreType.DMA,
            pltpu.SemaphoreType.DMA,
            pltpu.SemaphoreType.DMA,
        ],
    )
    def k(tab_hbm, xa_hbm, xb_hbm, out_hbm,
          xav, xbv, ii, buf0, buf1, gsem0, gsem1, wsem0, wsem1):
        wid = lax.axis_index("s") * _NC + lax.axis_index("c")
        wbase = wid * _PER_W
        bufs = (buf0, buf1)
        gsems = (gsem0, gsem1)
        wsems = (wsem0, wsem1)

        def compute_idx(ci, b):
            sl_h = pl.ds(wbase + ci * _CH, _CH)
            pltpu.sync_copy(xa_hbm.at[sl_h], xav.at[b])
            pltpu.sync_copy(xb_hbm.at[sl_h], xbv.at[b])
            for i in range(_CH // _LANES):
                sl = pl.ds(i * _LANES, _LANES)
                ii[b, sl] = _col(xav[b, sl]) * _N + _col(xbv[b, sl])

        def fire_gather(b):
            pltpu.async_copy(tab_hbm.at[ii.at[b]], bufs[b], gsems[b])

        def wait_gather(b):
            pltpu.make_async_copy(tab_hbm.at[ii.at[b]], bufs[b],
                                  gsems[b]).wait()

        def fire_write(ci, b):
            sl_h = pl.ds(wbase + ci * _CH, _CH)
            pltpu.async_copy(bufs[b], out_hbm.at[sl_h], wsems[b])

        def wait_write(ci, b):
            sl_h = pl.ds(wbase + ci * _CH, _CH)
            pltpu.make_async_copy(bufs[b], out_hbm.at[sl_h],
                                  wsems[b]).wait()

        compute_idx(0, 0)
        fire_gather(0)
        compute_idx(1, 1)
        fire_gather(1)

        @pl.loop(0, (_NCH - 2) // 2)
        def _steady(ci2):
            k2 = ci2 * 2
            for b in range(2):
                wait_gather(b)
                fire_write(k2 + b, b)
                compute_idx(k2 + 2 + b, b)
                wait_write(k2 + b, b)
                fire_gather(b)

        for b in range(2):
            wait_gather(b)
            fire_write(_NCH - 2 + b, b)
            wait_write(_NCH - 2 + b, b)

    return k(tab, xa, xb)


def _sc_gather3(x0, x1, x2, xyf, yzf, xzf):
    mesh = plsc.VectorSubcoreMesh(core_axis_name="c", subcore_axis_name="s")

    @functools.partial(
        pl.kernel,
        mesh=mesh,
        out_type=[
            jax.ShapeDtypeStruct((_B, _FP), jnp.float32),
            jax.ShapeDtypeStruct((_B, _FP), jnp.float32),
            jax.ShapeDtypeStruct((_B, _FP), jnp.float32),
        ],
        scratch_types=[
            pltpu.VMEM((2, _CH), jnp.float32),
            pltpu.VMEM((2, _CH), jnp.float32),
            pltpu.VMEM((2, _CH), jnp.float32),
            pltpu.VMEM((2, _CH), jnp.int32),
            pltpu.VMEM((2, _CH), jnp.int32),
            pltpu.VMEM((2, _CH), jnp.int32),
            pltpu.VMEM((_CH, _FP), jnp.float32),
            pltpu.VMEM((_CH, _FP), jnp.float32),
            pltpu.VMEM((_CH, _FP), jnp.float32),
            pltpu.VMEM((_CH, _FP), jnp.float32),
            pltpu.VMEM((_CH, _FP), jnp.float32),
            pltpu.VMEM((_CH, _FP), jnp.float32),
            pltpu.SemaphoreType.DMA,
            pltpu.SemaphoreType.DMA,
            pltpu.SemaphoreType.DMA,
            pltpu.SemaphoreType.DMA,
        ],
    )
    def k(xy_hbm, yz_hbm, xz_hbm, x0_hbm, x1_hbm, x2_hbm,
          oxy_hbm, oyz_hbm, oxz_hbm,
          x0v, x1v, x2v, ia, ib, ic,
          bA0, bB0, bC0, bA1, bB1, bC1, gsem0, gsem1, wsem0, wsem1):
        wid = lax.axis_index("s") * _NC + lax.axis_index("c")
        wbase = wid * _PER_W
        bufs = ((bA0, bB0, bC0), (bA1, bB1, bC1))
        gsems = (gsem0, gsem1)
        wsems = (wsem0, wsem1)
        tabs = (xy_hbm, yz_hbm, xz_hbm)

        def compute_idx(ci, b):
            sl_h = pl.ds(wbase + ci * _CH, _CH)
            pltpu.sync_copy(x0_hbm.at[sl_h], x0v.at[b])
            pltpu.sync_copy(x1_hbm.at[sl_h], x1v.at[b])
            pltpu.sync_copy(x2_hbm.at[sl_h], x2v.at[b])
            for i in range(_CH // _LANES):
                sl = pl.ds(i * _LANES, _LANES)
                c0 = _col(x0v[b, sl])
                c1 = _col(x1v[b, sl])
                c2 = _col(x2v[b, sl])
                ia[b, sl] = c0 * _N + c1
                ib[b, sl] = c1 * _N + c2
                ic[b, sl] = c0 * _N + c2

        def fire_gathers(b):
            idxs = (ia.at[b], ib.at[b], ic.at[b])
            return [pltpu.async_copy(tabs[p].at[idxs[p]], bufs[b][p],
                                     gsems[b])
                    for p in range(3)]

        def fire_writes(ci, b):
            sl_h = pl.ds(wbase + ci * _CH, _CH)
            outs = (oxy_hbm, oyz_hbm, oxz_hbm)
            return [pltpu.async_copy(bufs[b][p], outs[p].at[sl_h], wsems[b])
                    for p in range(3)]

        def wait_gathers(b):
            idxs = (ia.at[b], ib.at[b], ic.at[b])
            for p in range(3):
                pltpu.make_async_copy(tabs[p].at[idxs[p]], bufs[b][p],
                                      gsems[b]).wait()

        def wait_writes(ci, b):
            sl_h = pl.ds(wbase + ci * _CH, _CH)
            outs = (oxy_hbm, oyz_hbm, oxz_hbm)
            for p in range(3):
                pltpu.make_async_copy(bufs[b][p], outs[p].at[sl_h],
                                      wsems[b]).wait()

        compute_idx(0, 0)
        fire_gathers(0)
        compute_idx(1, 1)
        fire_gathers(1)

        @pl.loop(0, (_NCH - 2) // 2)
        def _steady(ci2):
            k2 = ci2 * 2
            for b in range(2):
                wait_gathers(b)
                fire_writes(k2 + b, b)
                compute_idx(k2 + 2 + b, b)
                wait_writes(k2 + b, b)
                fire_gathers(b)

        for b in range(2):
            wait_gathers(b)
            fire_writes(_NCH - 2 + b, b)
            wait_writes(_NCH - 2 + b, b)

    return k(xyf, yzf, xzf, x0, x1, x2)


def _tc_mlp(fxy, fyz, fxz, xT, dT, W1, b1, W2, b2, W3, b3, W4, b4, W5, b5):
    RS = np.zeros((3, 24), np.float32)
    for j in range(_L_DIR):
        for a in range(3):
            RS[a, 3 * j + a] = float(2 ** j)
            RS[a, 12 + 3 * j + a] = float(2 ** j)
    RSt = jnp.asarray(RS.T)
    sin_rows = np.concatenate(
        [np.arange(3 + 6 * j, 6 + 6 * j) for j in range(_L_DIR)])
    cos_rows = sin_rows + 3
    W1p = jnp.concatenate([W1, jnp.zeros((_FP - _F, _HID), jnp.float32)])
    W1t = W1p.T
    W2t = W2.T
    W3at = W3[0:3].T
    W3st = W3[jnp.asarray(sin_rows)].T
    W3ct = W3[jnp.asarray(cos_rows)].T
    W3ft = jnp.concatenate(
        [W3[27:42], jnp.zeros((1, _HID), jnp.float32)], axis=0).T
    W4t = W4.T
    W5t = W5.T
    b1c = b1.reshape(-1, 1)
    b2c = b2.reshape(-1, 1)
    b3c = b3.reshape(-1, 1)
    b4c = b4.reshape(-1, 1)
    b5c = b5.reshape(-1, 1)

    def body(fxy_r, fyz_r, fxz_r, xT_r, dT_r,
             W1r, b1r, W2r, b2r, RSr, W3ar, W3sr, W3cr, W3fr, b3r,
             W4r, b4r, W5r, b5r,
             c_r, s_r):
        ff = fxy_r[...] * fyz_r[...] * fxz_r[...]
        ffT = ff.T
        hT = jnp.maximum(jnp.dot(W1r[...], ffT) + b1r[...], 0.0)
        hT = jnp.maximum(jnp.dot(W2r[...], hT) + b2r[...], 0.0)
        ddT = dT_r[...]
        angT = jnp.dot(RSr[...], ddT)
        TsT = jnp.sin(angT[0:12])
        TcT = jnp.cos(angT[12:24])
        preT = (jnp.dot(W3ar[...], ddT) + jnp.dot(W3sr[...], TsT)
                + jnp.dot(W3cr[...], TcT) + jnp.dot(W3fr[...], hT)
                + b3r[...])
        h2T = jnp.maximum(preT, 0.0)
        h2T = jnp.maximum(jnp.dot(W4r[...], h2T) + b4r[...], 0.0)
        cT = jax.nn.sigmoid(jnp.dot(W5r[...], h2T) + b5r[...])
        xa = jnp.abs(xT_r[...])
        m = (xa[0:1, :] < _SCALE) & (xa[1:2, :] < _SCALE) & (xa[2:3, :] < _SCALE)
        mf = m.astype(jnp.float32)
        c_r[...] = cT * mf
        s_r[...] = hT[15:16, :] * mf

    feat_spec = pl.BlockSpec((_BB, _FP), lambda i: (i, 0))
    pt_spec = pl.BlockSpec((3, _BB), lambda i: (0, i))

    def full(a):
        return pl.BlockSpec(a.shape, lambda i: tuple(0 for _ in a.shape))

    weights = (W1t, b1c, W2t, b2c, RSt, W3at, W3st, W3ct, W3ft, b3c,
               W4t, b4c, W5t, b5c)
    cT, sigT = pl.pallas_call(
        body,
        grid=(_B // _BB,),
        in_specs=[feat_spec, feat_spec, feat_spec, pt_spec, pt_spec]
                 + [full(w) for w in weights],
        out_specs=[pl.BlockSpec((3, _BB), lambda i: (0, i)),
                   pl.BlockSpec((1, _BB), lambda i: (0, i))],
        out_shape=[jax.ShapeDtypeStruct((3, _B), jnp.float32),
                   jax.ShapeDtypeStruct((1, _B), jnp.float32)],
        compiler_params=pltpu.CompilerParams(
            dimension_semantics=("parallel",)),
    )(fxy, fyz, fxz, xT, dT, *weights)
    return cT, sigT


def kernel(x, d, xy_plane, yz_plane, xz_plane,
           W1, b1, W2, b2, W3, b3, W4, b4, W5, b5):
    x0 = x[:, 0]
    x1 = x[:, 1]
    x2 = x[:, 2]
    xyf = _tc_relayout(jnp.transpose(xy_plane, (0, 2, 1)))
    yzf = _tc_relayout(jnp.transpose(yz_plane, (0, 2, 1)))
    xzf = _tc_relayout(jnp.transpose(xz_plane, (0, 2, 1)))
    fxy = _sc_gather1(x0, x1, xyf)
    fyz = _sc_gather1(x1, x2, yzf)
    fxz = _sc_gather1(x0, x2, xzf)
    cT, sigT = _tc_mlp(fxy, fyz, fxz, x.T, d.T,
                       W1, b1, W2, b2, W3, b3, W4, b4, W5, b5)
    return cT.T, sigT.reshape(_B)


# speedup vs baseline: 1.0139x; 1.0139x over previous
import functools

import jax
import jax.numpy as jnp
import numpy as np
from jax import lax
from jax.experimental import pallas as pl
from jax.experimental.pallas import tpu as pltpu
from jax.experimental.pallas import tpu_sc as plsc

_B = 262144
_N = 512
_F = 96
_HID = 64
_L_DIR = 4
_SCALE = 1.5

_NC = 2
_NS = 16
_NW = _NC * _NS
_LANES = 16
_PER_W = _B // _NW
_FP = 128
_CH = 128
_NCH = _PER_W // _CH

_BB = 1024


def _col(v):
    q = (v / (2.0 * _SCALE) + 0.5) * float(_N)
    qi = q.astype(jnp.int32)
    return jnp.clip(qi, 0, _N - 1)


def _tc_relayout(planeT):
    def body(in_r, out_r):
        t = in_r[0]
        tt = t.T
        out_r[:, 0:_F] = tt
        out_r[:, _F:_FP] = jnp.zeros((_N, _FP - _F), jnp.float32)

    return pl.pallas_call(
        body,
        grid=(_N,),
        in_specs=[pl.BlockSpec((1, _F, _N), lambda i: (i, 0, 0))],
        out_specs=pl.BlockSpec((_N, _FP), lambda i: (i, 0)),
        out_shape=jax.ShapeDtypeStruct((_N * _N, _FP), jnp.float32),
        compiler_params=pltpu.CompilerParams(
            dimension_semantics=("parallel",)),
    )(planeT)


def _sc_gather1(xa, xb, tab):
    mesh = plsc.VectorSubcoreMesh(core_axis_name="c", subcore_axis_name="s")

    @functools.partial(
        pl.kernel,
        mesh=mesh,
        out_type=jax.ShapeDtypeStruct((_B, _FP), jnp.float32),
        scratch_types=[
            pltpu.VMEM((2, _CH), jnp.float32),
            pltpu.VMEM((2, _CH), jnp.float32),
            pltpu.VMEM((2, _CH), jnp.int32),
            pltpu.VMEM((_CH, _FP), jnp.float32),
            pltpu.VMEM((_CH, _FP), jnp.float32),
            pltpu.SemaphoreType.DMA,
            pltpu.SemaphoreType.DMA,
            pltpu.SemaphoreType.DMA,
            pltpu.SemaphoreType.DMA,
        ],
    )
    def k(tab_hbm, xa_hbm, xb_hbm, out_hbm,
          xav, xbv, ii, buf0, buf1, gsem0, gsem1, wsem0, wsem1):
        wid = lax.axis_index("s") * _NC + lax.axis_index("c")
        wbase = wid * _PER_W
        bufs = (buf0, buf1)
        gsems = (gsem0, gsem1)
        wsems = (wsem0, wsem1)

        def compute_idx(ci, b):
            sl_h = pl.ds(wbase + ci * _CH, _CH)
            pltpu.sync_copy(xa_hbm.at[sl_h], xav.at[b])
            pltpu.sync_copy(xb_hbm.at[sl_h], xbv.at[b])
            for i in range(_CH // _LANES):
                sl = pl.ds(i * _LANES, _LANES)
                ii[b, sl] = _col(xav[b, sl]) * _N + _col(xbv[b, sl])

        def fire_gather(b):
            pltpu.async_copy(tab_hbm.at[ii.at[b]], bufs[b], gsems[b])

        def wait_gather(b):
            pltpu.make_async_copy(tab_hbm.at[ii.at[b]], bufs[b],
                                  gsems[b]).wait()

        def fire_write(ci, b):
            sl_h = pl.ds(wbase + ci * _CH, _CH)
            pltpu.async_copy(bufs[b], out_hbm.at[sl_h], wsems[b])

        def wait_write(ci, b):
            sl_h = pl.ds(wbase + ci * _CH, _CH)
            pltpu.make_async_copy(bufs[b], out_hbm.at[sl_h],
                                  wsems[b]).wait()

        compute_idx(0, 0)
        fire_gather(0)
        compute_idx(1, 1)
        fire_gather(1)

        @pl.loop(0, (_NCH - 2) // 2)
        def _steady(ci2):
            k2 = ci2 * 2
            for b in range(2):
                wait_gather(b)
                fire_write(k2 + b, b)
                compute_idx(k2 + 2 + b, b)
                wait_write(k2 + b, b)
                fire_gather(b)

        for b in range(2):
            wait_gather(b)
            fire_write(_NCH - 2 + b, b)
            wait_write(_NCH - 2 + b, b)

    return k(tab, xa, xb)


def _sc_gather3(x0, x1, x2, xyf, yzf, xzf):
    mesh = plsc.VectorSubcoreMesh(core_axis_name="c", subcore_axis_name="s")

    @functools.partial(
        pl.kernel,
        mesh=mesh,
        out_type=[
            jax.ShapeDtypeStruct((_B, _FP), jnp.float32),
            jax.ShapeDtypeStruct((_B, _FP), jnp.float32),
            jax.ShapeDtypeStruct((_B, _FP), jnp.float32),
        ],
        scratch_types=[
            pltpu.VMEM((2, _CH), jnp.float32),
            pltpu.VMEM((2, _CH), jnp.float32),
            pltpu.VMEM((2, _CH), jnp.float32),
            pltpu.VMEM((2, _CH), jnp.int32),
            pltpu.VMEM((2, _CH), jnp.int32),
            pltpu.VMEM((2, _CH), jnp.int32),
            pltpu.VMEM((_CH, _FP), jnp.float32),
            pltpu.VMEM((_CH, _FP), jnp.float32),
            pltpu.VMEM((_CH, _FP), jnp.float32),
            pltpu.VMEM((_CH, _FP), jnp.float32),
            pltpu.VMEM((_CH, _FP), jnp.float32),
            pltpu.VMEM((_CH, _FP), jnp.float32),
            pltpu.SemaphoreType.DMA,
            pltpu.SemaphoreType.DMA,
            pltpu.SemaphoreType.DMA,
            pltpu.SemaphoreType.DMA,
        ],
    )
    def k(xy_hbm, yz_hbm, xz_hbm, x0_hbm, x1_hbm, x2_hbm,
          oxy_hbm, oyz_hbm, oxz_hbm,
          x0v, x1v, x2v, ia, ib, ic,
          bA0, bB0, bC0, bA1, bB1, bC1, gsem0, gsem1, wsem0, wsem1):
        wid = lax.axis_index("s") * _NC + lax.axis_index("c")
        wbase = wid * _PER_W
        bufs = ((bA0, bB0, bC0), (bA1, bB1, bC1))
        gsems = (gsem0, gsem1)
        wsems = (wsem0, wsem1)
        tabs = (xy_hbm, yz_hbm, xz_hbm)

        def compute_idx(ci, b):
            sl_h = pl.ds(wbase + ci * _CH, _CH)
            pltpu.sync_copy(x0_hbm.at[sl_h], x0v.at[b])
            pltpu.sync_copy(x1_hbm.at[sl_h], x1v.at[b])
            pltpu.sync_copy(x2_hbm.at[sl_h], x2v.at[b])
            for i in range(_CH // _LANES):
                sl = pl.ds(i * _LANES, _LANES)
                c0 = _col(x0v[b, sl])
                c1 = _col(x1v[b, sl])
                c2 = _col(x2v[b, sl])
                ia[b, sl] = c0 * _N + c1
                ib[b, sl] = c1 * _N + c2
                ic[b, sl] = c0 * _N + c2

        def fire_gathers(b):
            idxs = (ia.at[b], ib.at[b], ic.at[b])
            return [pltpu.async_copy(tabs[p].at[idxs[p]], bufs[b][p],
                                     gsems[b])
                    for p in range(3)]

        def fire_writes(ci, b):
            sl_h = pl.ds(wbase + ci * _CH, _CH)
            outs = (oxy_hbm, oyz_hbm, oxz_hbm)
            return [pltpu.async_copy(bufs[b][p], outs[p].at[sl_h], wsems[b])
                    for p in range(3)]

        def wait_gathers(b):
            idxs = (ia.at[b], ib.at[b], ic.at[b])
            for p in range(3):
                pltpu.make_async_copy(tabs[p].at[idxs[p]], bufs[b][p],
                                      gsems[b]).wait()

        def wait_writes(ci, b):
            sl_h = pl.ds(wbase + ci * _CH, _CH)
            outs = (oxy_hbm, oyz_hbm, oxz_hbm)
            for p in range(3):
                pltpu.make_async_copy(bufs[b][p], outs[p].at[sl_h],
                                      wsems[b]).wait()

        compute_idx(0, 0)
        fire_gathers(0)
        compute_idx(1, 1)
        fire_gathers(1)

        @pl.loop(0, (_NCH - 2) // 2)
        def _steady(ci2):
            k2 = ci2 * 2
            for b in range(2):
                wait_gathers(b)
                fire_writes(k2 + b, b)
                compute_idx(k2 + 2 + b, b)
                wait_writes(k2 + b, b)
                fire_gathers(b)

        for b in range(2):
            wait_gathers(b)
            fire_writes(_NCH - 2 + b, b)
            wait_writes(_NCH - 2 + b, b)

    return k(xyf, yzf, xzf, x0, x1, x2)


def _tc_mlp(fxy, fyz, fxz, xT, dT, W1, b1, W2, b2, W3, b3, W4, b4, W5, b5):
    RS = np.zeros((3, 24), np.float32)
    for j in range(_L_DIR):
        for a in range(3):
            RS[a, 3 * j + a] = float(2 ** j)
            RS[a, 12 + 3 * j + a] = float(2 ** j)
    RSt = jnp.asarray(RS.T)
    sin_rows = np.concatenate(
        [np.arange(3 + 6 * j, 6 + 6 * j) for j in range(_L_DIR)])
    cos_rows = sin_rows + 3
    W1p = jnp.concatenate([W1, jnp.zeros((_FP - _F, _HID), jnp.float32)])
    W1t = W1p.T
    W2t = W2.T
    W3at = W3[0:3].T
    W3st = W3[jnp.asarray(sin_rows)].T
    W3ct = W3[jnp.asarray(cos_rows)].T
    W3ft = jnp.concatenate(
        [W3[27:42], jnp.zeros((1, _HID), jnp.float32)], axis=0).T
    W4t = W4.T
    W5t = W5.T
    b1c = b1.reshape(-1, 1)
    b2c = b2.reshape(-1, 1)
    b3c = b3.reshape(-1, 1)
    b4c = b4.reshape(-1, 1)
    b5c = b5.reshape(-1, 1)

    def body(fxy_r, fyz_r, fxz_r, xT_r, dT_r,
             W1r, b1r, W2r, b2r, RSr, W3ar, W3sr, W3cr, W3fr, b3r,
             W4r, b4r, W5r, b5r,
             c_r, s_r):
        ff = fxy_r[...] * fyz_r[...] * fxz_r[...]
        ffT = ff.T
        hT = jnp.maximum(jnp.dot(W1r[...], ffT) + b1r[...], 0.0)
        hT = jnp.maximum(jnp.dot(W2r[...], hT) + b2r[...], 0.0)
        ddT = dT_r[...]
        angT = jnp.dot(RSr[...], ddT)
        TsT = jnp.sin(angT[0:12])
        TcT = jnp.cos(angT[12:24])
        preT = (jnp.dot(W3ar[...], ddT) + jnp.dot(W3sr[...], TsT)
                + jnp.dot(W3cr[...], TcT) + jnp.dot(W3fr[...], hT)
                + b3r[...])
        h2T = jnp.maximum(preT, 0.0)
        h2T = jnp.maximum(jnp.dot(W4r[...], h2T) + b4r[...], 0.0)
        cT = jax.nn.sigmoid(jnp.dot(W5r[...], h2T) + b5r[...])
        xa = jnp.abs(xT_r[...])
        m = (xa[0:1, :] < _SCALE) & (xa[1:2, :] < _SCALE) & (xa[2:3, :] < _SCALE)
        mf = m.astype(jnp.float32)
        c_r[...] = cT * mf
        s_r[...] = hT[15:16, :] * mf

    feat_spec = pl.BlockSpec((_BB, _FP), lambda i: (i, 0))
    pt_spec = pl.BlockSpec((3, _BB), lambda i: (0, i))

    def full(a):
        return pl.BlockSpec(a.shape, lambda i: tuple(0 for _ in a.shape))

    weights = (W1t, b1c, W2t, b2c, RSt, W3at, W3st, W3ct, W3ft, b3c,
               W4t, b4c, W5t, b5c)
    cT, sigT = pl.pallas_call(
        body,
        grid=(_B // _BB,),
        in_specs=[feat_spec, feat_spec, feat_spec, pt_spec, pt_spec]
                 + [full(w) for w in weights],
        out_specs=[pl.BlockSpec((3, _BB), lambda i: (0, i)),
                   pl.BlockSpec((1, _BB), lambda i: (0, i))],
        out_shape=[jax.ShapeDtypeStruct((3, _B), jnp.float32),
                   jax.ShapeDtypeStruct((1, _B), jnp.float32)],
        compiler_params=pltpu.CompilerParams(
            dimension_semantics=("parallel",)),
    )(fxy, fyz, fxz, xT, dT, *weights)
    return cT, sigT


def kernel(x, d, xy_plane, yz_plane, xz_plane,
           W1, b1, W2, b2, W3, b3, W4, b4, W5, b5):
    x0 = x[:, 0]
    x1 = x[:, 1]
    x2 = x[:, 2]
    xyf = _tc_relayout(jnp.transpose(xy_plane, (0, 2, 1)))
    yzf = _tc_relayout(jnp.transpose(yz_plane, (0, 2, 1)))
    xzf = _tc_relayout(jnp.transpose(xz_plane, (0, 2, 1)))
    fxy, fyz, fxz = _sc_gather3(x0, x1, x2, xyf, yzf, xzf)
    cT, sigT = _tc_mlp(fxy, fyz, fxz, x.T, d.T,
                       W1, b1, W2, b2, W3, b3, W4, b4, W5, b5)
    return cT.T, sigT.reshape(_B)


# speedup vs baseline: 1.5581x; 1.5368x over previous
import functools

import jax
import jax.numpy as jnp
import numpy as np
from jax import lax
from jax.experimental import pallas as pl
from jax.experimental.pallas import tpu as pltpu
from jax.experimental.pallas import tpu_sc as plsc

_B = 262144
_N = 512
_F = 96
_HID = 64
_L_DIR = 4
_SCALE = 1.5

_NC = 2
_NS = 16
_NW = _NC * _NS
_LANES = 16
_PER_W = _B // _NW
_FP = 128
_CH = 128
_NCH = _PER_W // _CH

_BB = 2048


def _col(v):
    q = (v / (2.0 * _SCALE) + 0.5) * float(_N)
    qi = q.astype(jnp.int32)
    return jnp.clip(qi, 0, _N - 1)


def _tc_relayout(planeT):
    def body(in_r, out_r):
        t = in_r[0]
        tt = t.T
        out_r[:, 0:_F] = tt
        out_r[:, _F:_FP] = jnp.zeros((_N, _FP - _F), jnp.float32)

    return pl.pallas_call(
        body,
        grid=(_N,),
        in_specs=[pl.BlockSpec((1, _F, _N), lambda i: (i, 0, 0))],
        out_specs=pl.BlockSpec((_N, _FP), lambda i: (i, 0)),
        out_shape=jax.ShapeDtypeStruct((_N * _N, _FP), jnp.float32),
        compiler_params=pltpu.CompilerParams(
            dimension_semantics=("parallel",)),
    )(planeT)


def _sc_gather1(xa, xb, tab):
    mesh = plsc.VectorSubcoreMesh(core_axis_name="c", subcore_axis_name="s")

    @functools.partial(
        pl.kernel,
        mesh=mesh,
        out_type=jax.ShapeDtypeStruct((_B, _FP), jnp.float32),
        scratch_types=[
            pltpu.VMEM((2, _CH), jnp.float32),
            pltpu.VMEM((2, _CH), jnp.float32),
            pltpu.VMEM((2, _CH), jnp.int32),
            pltpu.VMEM((_CH, _FP), jnp.float32),
            pltpu.VMEM((_CH, _FP), jnp.float32),
            pltpu.SemaphoreType.DMA,
            pltpu.SemaphoreType.DMA,
            pltpu.SemaphoreType.DMA,
            pltpu.SemaphoreType.DMA,
        ],
    )
    def k(tab_hbm, xa_hbm, xb_hbm, out_hbm,
          xav, xbv, ii, buf0, buf1, gsem0, gsem1, wsem0, wsem1):
        wid = lax.axis_index("s") * _NC + lax.axis_index("c")
        wbase = wid * _PER_W
        bufs = (buf0, buf1)
        gsems = (gsem0, gsem1)
        wsems = (wsem0, wsem1)

        def compute_idx(ci, b):
            sl_h = pl.ds(wbase + ci * _CH, _CH)
            pltpu.sync_copy(xa_hbm.at[sl_h], xav.at[b])
            pltpu.sync_copy(xb_hbm.at[sl_h], xbv.at[b])
            for i in range(_CH // _LANES):
                sl = pl.ds(i * _LANES, _LANES)
                ii[b, sl] = _col(xav[b, sl]) * _N + _col(xbv[b, sl])

        def fire_gather(b):
            pltpu.async_copy(tab_hbm.at[ii.at[b]], bufs[b], gsems[b])

        def wait_gather(b):
            pltpu.make_async_copy(tab_hbm.at[ii.at[b]], bufs[b],
                                  gsems[b]).wait()

        def fire_write(ci, b):
            sl_h = pl.ds(wbase + ci * _CH, _CH)
            pltpu.async_copy(bufs[b], out_hbm.at[sl_h], wsems[b])

        def wait_write(ci, b):
            sl_h = pl.ds(wbase + ci * _CH, _CH)
            pltpu.make_async_copy(bufs[b], out_hbm.at[sl_h],
                                  wsems[b]).wait()

        compute_idx(0, 0)
        fire_gather(0)
        compute_idx(1, 1)
        fire_gather(1)

        @pl.loop(0, (_NCH - 2) // 2)
        def _steady(ci2):
            k2 = ci2 * 2
            for b in range(2):
                wait_gather(b)
                fire_write(k2 + b, b)
                compute_idx(k2 + 2 + b, b)
                wait_write(k2 + b, b)
                fire_gather(b)

        for b in range(2):
            wait_gather(b)
            fire_write(_NCH - 2 + b, b)
            wait_write(_NCH - 2 + b, b)

    return k(tab, xa, xb)


def _sc_gather3(x0, x1, x2, xyf, yzf, xzf):
    mesh = plsc.VectorSubcoreMesh(core_axis_name="c", subcore_axis_name="s")

    @functools.partial(
        pl.kernel,
        mesh=mesh,
        out_type=[
            jax.ShapeDtypeStruct((_B, _FP), jnp.float32),
            jax.ShapeDtypeStruct((_B, _FP), jnp.float32),
            jax.ShapeDtypeStruct((_B, _FP), jnp.float32),
        ],
        scratch_types=[
            pltpu.VMEM((2, _CH), jnp.float32),
            pltpu.VMEM((2, _CH), jnp.float32),
            pltpu.VMEM((2, _CH), jnp.float32),
            pltpu.VMEM((2, _CH), jnp.int32),
            pltpu.VMEM((2, _CH), jnp.int32),
            pltpu.VMEM((2, _CH), jnp.int32),
            pltpu.VMEM((_CH, _FP), jnp.float32),
            pltpu.VMEM((_CH, _FP), jnp.float32),
            pltpu.VMEM((_CH, _FP), jnp.float32),
            pltpu.VMEM((_CH, _FP), jnp.float32),
            pltpu.VMEM((_CH, _FP), jnp.float32),
            pltpu.VMEM((_CH, _FP), jnp.float32),
            pltpu.SemaphoreType.DMA,
            pltpu.SemaphoreType.DMA,
            pltpu.SemaphoreType.DMA,
            pltpu.SemaphoreType.DMA,
        ],
    )
    def k(xy_hbm, yz_hbm, xz_hbm, x0_hbm, x1_hbm, x2_hbm,
          oxy_hbm, oyz_hbm, oxz_hbm,
          x0v, x1v, x2v, ia, ib, ic,
          bA0, bB0, bC0, bA1, bB1, bC1, gsem0, gsem1, wsem0, wsem1):
        wid = lax.axis_index("s") * _NC + lax.axis_index("c")
        wbase = wid * _PER_W
        bufs = ((bA0, bB0, bC0), (bA1, bB1, bC1))
        gsems = (gsem0, gsem1)
        wsems = (wsem0, wsem1)
        tabs = (xy_hbm, yz_hbm, xz_hbm)

        def compute_idx(ci, b):
            sl_h = pl.ds(wbase + ci * _CH, _CH)
            pltpu.sync_copy(x0_hbm.at[sl_h], x0v.at[b])
            pltpu.sync_copy(x1_hbm.at[sl_h], x1v.at[b])
            pltpu.sync_copy(x2_hbm.at[sl_h], x2v.at[b])
            for i in range(_CH // _LANES):
                sl = pl.ds(i * _LANES, _LANES)
                c0 = _col(x0v[b, sl])
                c1 = _col(x1v[b, sl])
                c2 = _col(x2v[b, sl])
                ia[b, sl] = c0 * _N + c1
                ib[b, sl] = c1 * _N + c2
                ic[b, sl] = c0 * _N + c2

        def fire_gathers(b):
            idxs = (ia.at[b], ib.at[b], ic.at[b])
            return [pltpu.async_copy(tabs[p].at[idxs[p]], bufs[b][p],
                                     gsems[b])
                    for p in range(3)]

        def fire_writes(ci, b):
            sl_h = pl.ds(wbase + ci * _CH, _CH)
            outs = (oxy_hbm, oyz_hbm, oxz_hbm)
            return [pltpu.async_copy(bufs[b][p], outs[p].at[sl_h], wsems[b])
                    for p in range(3)]

        def wait_gathers(b):
            idxs = (ia.at[b], ib.at[b], ic.at[b])
            for p in range(3):
                pltpu.make_async_copy(tabs[p].at[idxs[p]], bufs[b][p],
                                      gsems[b]).wait()

        def wait_writes(ci, b):
            sl_h = pl.ds(wbase + ci * _CH, _CH)
            outs = (oxy_hbm, oyz_hbm, oxz_hbm)
            for p in range(3):
                pltpu.make_async_copy(bufs[b][p], outs[p].at[sl_h],
                                      wsems[b]).wait()

        compute_idx(0, 0)
        fire_gathers(0)
        compute_idx(1, 1)
        fire_gathers(1)

        @pl.loop(0, (_NCH - 2) // 2)
        def _steady(ci2):
            k2 = ci2 * 2
            for b in range(2):
                wait_gathers(b)
                fire_writes(k2 + b, b)
                compute_idx(k2 + 2 + b, b)
                wait_writes(k2 + b, b)
                fire_gathers(b)

        for b in range(2):
            wait_gathers(b)
            fire_writes(_NCH - 2 + b, b)
            wait_writes(_NCH - 2 + b, b)

    return k(xyf, yzf, xzf, x0, x1, x2)


def _tc_mlp(fxy, fyz, fxz, xT, dT, W1, b1, W2, b2, W3, b3, W4, b4, W5, b5):
    RS = np.zeros((3, 24), np.float32)
    for j in range(_L_DIR):
        for a in range(3):
            RS[a, 3 * j + a] = float(2 ** j)
            RS[a, 12 + 3 * j + a] = float(2 ** j)
    RSt = jnp.asarray(RS.T)
    sin_rows = np.concatenate(
        [np.arange(3 + 6 * j, 6 + 6 * j) for j in range(_L_DIR)])
    cos_rows = sin_rows + 3
    W1p = jnp.concatenate([W1, jnp.zeros((_FP - _F, _HID), jnp.float32)])
    W1t = W1p.T.astype(jnp.bfloat16)
    W2t = W2.T.astype(jnp.bfloat16)
    W3at = W3[0:3].T
    W3st = W3[jnp.asarray(sin_rows)].T.astype(jnp.bfloat16)
    W3ct = W3[jnp.asarray(cos_rows)].T.astype(jnp.bfloat16)
    W3ft = jnp.concatenate(
        [W3[27:42], jnp.zeros((1, _HID), jnp.float32)],
        axis=0).T.astype(jnp.bfloat16)
    W4t = W4.T.astype(jnp.bfloat16)
    W5t = W5.T.astype(jnp.bfloat16)
    b1c = b1.reshape(-1, 1)
    b2c = b2.reshape(-1, 1)
    b3c = b3.reshape(-1, 1)
    b4c = b4.reshape(-1, 1)
    b5c = b5.reshape(-1, 1)

    def body(fxy_r, fyz_r, fxz_r, xT_r, dT_r,
             W1r, b1r, W2r, b2r, RSr, W3ar, W3sr, W3cr, W3fr, b3r,
             W4r, b4r, W5r, b5r,
             c_r, s_r):
        f32 = jnp.float32
        bf16 = jnp.bfloat16

        def mm(w, a):
            return jnp.dot(w, a.astype(bf16), preferred_element_type=f32)

        ff = fxy_r[...] * fyz_r[...] * fxz_r[...]
        ffT = ff.T
        hT = jnp.maximum(mm(W1r[...], ffT) + b1r[...], 0.0)
        hT = jnp.maximum(mm(W2r[...], hT) + b2r[...], 0.0)
        ddT = dT_r[...]
        angT = jnp.dot(RSr[...], ddT)
        TsT = jnp.sin(angT[0:12])
        TcT = jnp.cos(angT[12:24])
        preT = (jnp.dot(W3ar[...], ddT) + mm(W3sr[...], TsT)
                + mm(W3cr[...], TcT) + mm(W3fr[...], hT)
                + b3r[...])
        h2T = jnp.maximum(preT, 0.0)
        h2T = jnp.maximum(mm(W4r[...], h2T) + b4r[...], 0.0)
        cT = jax.nn.sigmoid(mm(W5r[...], h2T) + b5r[...])
        xa = jnp.abs(xT_r[...])
        m = (xa[0:1, :] < _SCALE) & (xa[1:2, :] < _SCALE) & (xa[2:3, :] < _SCALE)
        mf = m.astype(jnp.float32)
        c_r[...] = cT * mf
        s_r[...] = hT[15:16, :] * mf

    feat_spec = pl.BlockSpec((_BB, _FP), lambda i: (i, 0))
    pt_spec = pl.BlockSpec((3, _BB), lambda i: (0, i))

    def full(a):
        return pl.BlockSpec(a.shape, lambda i: tuple(0 for _ in a.shape))

    weights = (W1t, b1c, W2t, b2c, RSt, W3at, W3st, W3ct, W3ft, b3c,
               W4t, b4c, W5t, b5c)
    cT, sigT = pl.pallas_call(
        body,
        grid=(_B // _BB,),
        in_specs=[feat_spec, feat_spec, feat_spec, pt_spec, pt_spec]
                 + [full(w) for w in weights],
        out_specs=[pl.BlockSpec((3, _BB), lambda i: (0, i)),
                   pl.BlockSpec((1, _BB), lambda i: (0, i))],
        out_shape=[jax.ShapeDtypeStruct((3, _B), jnp.float32),
                   jax.ShapeDtypeStruct((1, _B), jnp.float32)],
        compiler_params=pltpu.CompilerParams(
            dimension_semantics=("parallel",)),
    )(fxy, fyz, fxz, xT, dT, *weights)
    return cT, sigT


def kernel(x, d, xy_plane, yz_plane, xz_plane,
           W1, b1, W2, b2, W3, b3, W4, b4, W5, b5):
    x0 = x[:, 0]
    x1 = x[:, 1]
    x2 = x[:, 2]
    pad = ((0, 0), (0, _FP - _F))
    xyf = jnp.pad(xy_plane.reshape(_N * _N, _F), pad)
    yzf = jnp.pad(yz_plane.reshape(_N * _N, _F), pad)
    xzf = jnp.pad(xz_plane.reshape(_N * _N, _F), pad)
    fxy, fyz, fxz = _sc_gather3(x0, x1, x2, xyf, yzf, xzf)
    cT, sigT = _tc_mlp(fxy, fyz, fxz, x.T, d.T,
                       W1, b1, W2, b2, W3, b3, W4, b4, W5, b5)
    return cT.T, sigT.reshape(_B)


# speedup vs baseline: 1.6232x; 1.0418x over previous
import functools

import jax
import jax.numpy as jnp
import numpy as np
from jax import lax
from jax.experimental import pallas as pl
from jax.experimental.pallas import tpu as pltpu
from jax.experimental.pallas import tpu_sc as plsc

_B = 262144
_N = 512
_F = 96
_HID = 64
_L_DIR = 4
_SCALE = 1.5

_NC = 2
_NS = 16
_NW = _NC * _NS
_LANES = 16
_PER_W = _B // _NW
_FP = 128
_CH = 128
_NCH = _PER_W // _CH

_BB = 2048
_NHALF = 2


def _col(v):
    q = (v / (2.0 * _SCALE) + 0.5) * float(_N)
    qi = q.astype(jnp.int32)
    return jnp.clip(qi, 0, _N - 1)


def _tc_relayout(planeT):
    def body(in_r, out_r):
        t = in_r[0]
        tt = t.T
        out_r[:, 0:_F] = tt
        out_r[:, _F:_FP] = jnp.zeros((_N, _FP - _F), jnp.float32)

    return pl.pallas_call(
        body,
        grid=(_N,),
        in_specs=[pl.BlockSpec((1, _F, _N), lambda i: (i, 0, 0))],
        out_specs=pl.BlockSpec((_N, _FP), lambda i: (i, 0)),
        out_shape=jax.ShapeDtypeStruct((_N * _N, _FP), jnp.float32),
        compiler_params=pltpu.CompilerParams(
            dimension_semantics=("parallel",)),
    )(planeT)


def _sc_gather1(xa, xb, tab):
    mesh = plsc.VectorSubcoreMesh(core_axis_name="c", subcore_axis_name="s")

    @functools.partial(
        pl.kernel,
        mesh=mesh,
        out_type=jax.ShapeDtypeStruct((_B, _FP), jnp.float32),
        scratch_types=[
            pltpu.VMEM((2, _CH), jnp.float32),
            pltpu.VMEM((2, _CH), jnp.float32),
            pltpu.VMEM((2, _CH), jnp.int32),
            pltpu.VMEM((_CH, _FP), jnp.float32),
            pltpu.VMEM((_CH, _FP), jnp.float32),
            pltpu.SemaphoreType.DMA,
            pltpu.SemaphoreType.DMA,
            pltpu.SemaphoreType.DMA,
            pltpu.SemaphoreType.DMA,
        ],
    )
    def k(tab_hbm, xa_hbm, xb_hbm, out_hbm,
          xav, xbv, ii, buf0, buf1, gsem0, gsem1, wsem0, wsem1):
        wid = lax.axis_index("s") * _NC + lax.axis_index("c")
        wbase = wid * _PER_W
        bufs = (buf0, buf1)
        gsems = (gsem0, gsem1)
        wsems = (wsem0, wsem1)

        def compute_idx(ci, b):
            sl_h = pl.ds(wbase + ci * _CH, _CH)
            pltpu.sync_copy(xa_hbm.at[sl_h], xav.at[b])
            pltpu.sync_copy(xb_hbm.at[sl_h], xbv.at[b])
            for i in range(_CH // _LANES):
                sl = pl.ds(i * _LANES, _LANES)
                ii[b, sl] = _col(xav[b, sl]) * _N + _col(xbv[b, sl])

        def fire_gather(b):
            pltpu.async_copy(tab_hbm.at[ii.at[b]], bufs[b], gsems[b])

        def wait_gather(b):
            pltpu.make_async_copy(tab_hbm.at[ii.at[b]], bufs[b],
                                  gsems[b]).wait()

        def fire_write(ci, b):
            sl_h = pl.ds(wbase + ci * _CH, _CH)
            pltpu.async_copy(bufs[b], out_hbm.at[sl_h], wsems[b])

        def wait_write(ci, b):
            sl_h = pl.ds(wbase + ci * _CH, _CH)
            pltpu.make_async_copy(bufs[b], out_hbm.at[sl_h],
                                  wsems[b]).wait()

        compute_idx(0, 0)
        fire_gather(0)
        compute_idx(1, 1)
        fire_gather(1)

        @pl.loop(0, (_NCH - 2) // 2)
        def _steady(ci2):
            k2 = ci2 * 2
            for b in range(2):
                wait_gather(b)
                fire_write(k2 + b, b)
                compute_idx(k2 + 2 + b, b)
                wait_write(k2 + b, b)
                fire_gather(b)

        for b in range(2):
            wait_gather(b)
            fire_write(_NCH - 2 + b, b)
            wait_write(_NCH - 2 + b, b)

    return k(tab, xa, xb)


def _sc_gather3(npts, x0, x1, x2, xyf, yzf, xzf):
    mesh = plsc.VectorSubcoreMesh(core_axis_name="c", subcore_axis_name="s")

    per_w = npts // _NW
    nch = per_w // _CH

    @functools.partial(
        pl.kernel,
        mesh=mesh,
        out_type=[
            jax.ShapeDtypeStruct((npts, _FP), jnp.float32),
            jax.ShapeDtypeStruct((npts, _FP), jnp.float32),
            jax.ShapeDtypeStruct((npts, _FP), jnp.float32),
        ],
        scratch_types=[
            pltpu.VMEM((2, _CH), jnp.float32),
            pltpu.VMEM((2, _CH), jnp.float32),
            pltpu.VMEM((2, _CH), jnp.float32),
            pltpu.VMEM((2, _CH), jnp.int32),
            pltpu.VMEM((2, _CH), jnp.int32),
            pltpu.VMEM((2, _CH), jnp.int32),
            pltpu.VMEM((_CH, _FP), jnp.float32),
            pltpu.VMEM((_CH, _FP), jnp.float32),
            pltpu.VMEM((_CH, _FP), jnp.float32),
            pltpu.VMEM((_CH, _FP), jnp.float32),
            pltpu.VMEM((_CH, _FP), jnp.float32),
            pltpu.VMEM((_CH, _FP), jnp.float32),
            pltpu.SemaphoreType.DMA,
            pltpu.SemaphoreType.DMA,
            pltpu.SemaphoreType.DMA,
            pltpu.SemaphoreType.DMA,
        ],
    )
    def k(xy_hbm, yz_hbm, xz_hbm, x0_hbm, x1_hbm, x2_hbm,
          oxy_hbm, oyz_hbm, oxz_hbm,
          x0v, x1v, x2v, ia, ib, ic,
          bA0, bB0, bC0, bA1, bB1, bC1, gsem0, gsem1, wsem0, wsem1):
        wid = lax.axis_index("s") * _NC + lax.axis_index("c")
        wbase = wid * per_w
        bufs = ((bA0, bB0, bC0), (bA1, bB1, bC1))
        gsems = (gsem0, gsem1)
        wsems = (wsem0, wsem1)
        tabs = (xy_hbm, yz_hbm, xz_hbm)

        def compute_idx(ci, b):
            sl_h = pl.ds(wbase + ci * _CH, _CH)
            pltpu.sync_copy(x0_hbm.at[sl_h], x0v.at[b])
            pltpu.sync_copy(x1_hbm.at[sl_h], x1v.at[b])
            pltpu.sync_copy(x2_hbm.at[sl_h], x2v.at[b])
            for i in range(_CH // _LANES):
                sl = pl.ds(i * _LANES, _LANES)
                c0 = _col(x0v[b, sl])
                c1 = _col(x1v[b, sl])
                c2 = _col(x2v[b, sl])
                ia[b, sl] = c0 * _N + c1
                ib[b, sl] = c1 * _N + c2
                ic[b, sl] = c0 * _N + c2

        def fire_gathers(b):
            idxs = (ia.at[b], ib.at[b], ic.at[b])
            return [pltpu.async_copy(tabs[p].at[idxs[p]], bufs[b][p],
                                     gsems[b])
                    for p in range(3)]

        def fire_writes(ci, b):
            sl_h = pl.ds(wbase + ci * _CH, _CH)
            outs = (oxy_hbm, oyz_hbm, oxz_hbm)
            return [pltpu.async_copy(bufs[b][p], outs[p].at[sl_h], wsems[b])
                    for p in range(3)]

        def wait_gathers(b):
            idxs = (ia.at[b], ib.at[b], ic.at[b])
            for p in range(3):
                pltpu.make_async_copy(tabs[p].at[idxs[p]], bufs[b][p],
                                      gsems[b]).wait()

        def wait_writes(ci, b):
            sl_h = pl.ds(wbase + ci * _CH, _CH)
            outs = (oxy_hbm, oyz_hbm, oxz_hbm)
            for p in range(3):
                pltpu.make_async_copy(bufs[b][p], outs[p].at[sl_h],
                                      wsems[b]).wait()

        compute_idx(0, 0)
        fire_gathers(0)
        compute_idx(1, 1)
        fire_gathers(1)

        @pl.loop(0, (nch - 2) // 2)
        def _steady(ci2):
            k2 = ci2 * 2
            for b in range(2):
                wait_gathers(b)
                fire_writes(k2 + b, b)
                compute_idx(k2 + 2 + b, b)
                wait_writes(k2 + b, b)
                fire_gathers(b)

        for b in range(2):
            wait_gathers(b)
            fire_writes(nch - 2 + b, b)
            wait_writes(nch - 2 + b, b)

    return k(xyf, yzf, xzf, x0, x1, x2)


def _tc_mlp(npts, fxy, fyz, fxz, xT, dT, W1, b1, W2, b2, W3, b3, W4, b4, W5, b5):
    RS = np.zeros((3, 24), np.float32)
    for j in range(_L_DIR):
        for a in range(3):
            RS[a, 3 * j + a] = float(2 ** j)
            RS[a, 12 + 3 * j + a] = float(2 ** j)
    RSt = jnp.asarray(RS.T)
    sin_rows = np.concatenate(
        [np.arange(3 + 6 * j, 6 + 6 * j) for j in range(_L_DIR)])
    cos_rows = sin_rows + 3
    W1p = jnp.concatenate([W1, jnp.zeros((_FP - _F, _HID), jnp.float32)])
    W1t = W1p.T.astype(jnp.bfloat16)
    W2t = W2.T.astype(jnp.bfloat16)
    W3at = W3[0:3].T
    W3st = W3[jnp.asarray(sin_rows)].T.astype(jnp.bfloat16)
    W3ct = W3[jnp.asarray(cos_rows)].T.astype(jnp.bfloat16)
    W3ft = jnp.concatenate(
        [W3[27:42], jnp.zeros((1, _HID), jnp.float32)],
        axis=0).T.astype(jnp.bfloat16)
    W4t = W4.T.astype(jnp.bfloat16)
    W5t = W5.T.astype(jnp.bfloat16)
    b1c = b1.reshape(-1, 1)
    b2c = b2.reshape(-1, 1)
    b3c = b3.reshape(-1, 1)
    b4c = b4.reshape(-1, 1)
    b5c = b5.reshape(-1, 1)

    def body(fxy_r, fyz_r, fxz_r, xT_r, dT_r,
             W1r, b1r, W2r, b2r, RSr, W3ar, W3sr, W3cr, W3fr, b3r,
             W4r, b4r, W5r, b5r,
             c_r, s_r):
        f32 = jnp.float32
        bf16 = jnp.bfloat16

        def mm(w, a):
            return jnp.dot(w, a.astype(bf16), preferred_element_type=f32)

        ff = fxy_r[...] * fyz_r[...] * fxz_r[...]
        ffT = ff.T
        hT = jnp.maximum(mm(W1r[...], ffT) + b1r[...], 0.0)
        hT = jnp.maximum(mm(W2r[...], hT) + b2r[...], 0.0)
        ddT = dT_r[...]
        angT = jnp.dot(RSr[...], ddT)
        TsT = jnp.sin(angT[0:12])
        TcT = jnp.cos(angT[12:24])
        preT = (jnp.dot(W3ar[...], ddT) + mm(W3sr[...], TsT)
                + mm(W3cr[...], TcT) + mm(W3fr[...], hT)
                + b3r[...])
        h2T = jnp.maximum(preT, 0.0)
        h2T = jnp.maximum(mm(W4r[...], h2T) + b4r[...], 0.0)
        cT = jax.nn.sigmoid(mm(W5r[...], h2T) + b5r[...])
        xa = jnp.abs(xT_r[...])
        m = (xa[0:1, :] < _SCALE) & (xa[1:2, :] < _SCALE) & (xa[2:3, :] < _SCALE)
        mf = m.astype(jnp.float32)
        c_r[...] = cT * mf
        s_r[...] = hT[15:16, :] * mf

    feat_spec = pl.BlockSpec((_BB, _FP), lambda i: (i, 0))
    pt_spec = pl.BlockSpec((3, _BB), lambda i: (0, i))

    def full(a):
        return pl.BlockSpec(a.shape, lambda i: tuple(0 for _ in a.shape))

    weights = (W1t, b1c, W2t, b2c, RSt, W3at, W3st, W3ct, W3ft, b3c,
               W4t, b4c, W5t, b5c)
    cT, sigT = pl.pallas_call(
        body,
        grid=(npts // _BB,),
        in_specs=[feat_spec, feat_spec, feat_spec, pt_spec, pt_spec]
                 + [full(w) for w in weights],
        out_specs=[pl.BlockSpec((3, _BB), lambda i: (0, i)),
                   pl.BlockSpec((1, _BB), lambda i: (0, i))],
        out_shape=[jax.ShapeDtypeStruct((3, npts), jnp.float32),
                   jax.ShapeDtypeStruct((1, npts), jnp.float32)],
        compiler_params=pltpu.CompilerParams(
            dimension_semantics=("parallel",)),
    )(fxy, fyz, fxz, xT, dT, *weights)
    return cT, sigT


def kernel(x, d, xy_plane, yz_plane, xz_plane,
           W1, b1, W2, b2, W3, b3, W4, b4, W5, b5):
    x0 = x[:, 0]
    x1 = x[:, 1]
    x2 = x[:, 2]
    xT = x.T
    dT = d.T
    pad = ((0, 0), (0, _FP - _F))
    xyf = jnp.pad(xy_plane.reshape(_N * _N, _F), pad)
    yzf = jnp.pad(yz_plane.reshape(_N * _N, _F), pad)
    xzf = jnp.pad(xz_plane.reshape(_N * _N, _F), pad)
    nh = _B // _NHALF
    cts, sts = [], []
    for h in range(_NHALF):
        sl = slice(h * nh, (h + 1) * nh)
        f1, f2, f3 = _sc_gather3(nh, x0[sl], x1[sl], x2[sl], xyf, yzf, xzf)
        ct, st = _tc_mlp(nh, f1, f2, f3, xT[:, sl], dT[:, sl],
                         W1, b1, W2, b2, W3, b3, W4, b4, W5, b5)
        cts.append(ct)
        sts.append(st)
    cT = jnp.concatenate(cts, axis=1)
    sigT = jnp.concatenate(sts, axis=1)
    return cT.T, sigT.reshape(_B)
